# baseline (speedup 1.0000x reference)
import functools

import jax
import jax.numpy as jnp
from jax import lax
from jax.experimental import pallas as pl
from jax.experimental.pallas import tpu as pltpu
from jax.experimental.pallas import tpu_sc as plsc

ROWS, COLS = 4096, 50
D = 768
V = 64
NC, NS = 2, 16
NW = NC * NS
B_TOTAL = ROWS * COLS
B_PER_W = B_TOTAL // NW
CHUNK = 32
N_CHUNKS = B_PER_W // CHUNK
NBUF = 4

_mesh = plsc.VectorSubcoreMesh(core_axis_name="c", subcore_axis_name="s")


@functools.partial(
    pl.kernel,
    mesh=_mesh,
    out_type=jax.ShapeDtypeStruct((B_TOTAL, D), jnp.float32),
    scratch_types=[
        pltpu.VMEM((N_CHUNKS, CHUNK), jnp.int32),
        pltpu.VMEM((NBUF, CHUNK, D), jnp.float32),
        pltpu.SemaphoreType.DMA(NBUF),
        pltpu.SemaphoreType.DMA(NBUF),
    ],
)
def _embed(table_hbm, idx_hbm, out_hbm, idx_v, ring, sem_g, sem_s):
    cid = lax.axis_index("c")
    sid = lax.axis_index("s")
    wid = sid * NC + cid
    base = wid * B_PER_W

    pltpu.sync_copy(idx_hbm.at[wid], idx_v)

    def gather(n, b):
        return pltpu.async_copy(table_hbm.at[idx_v.at[n]], ring.at[b],
                                sem_g.at[b])

    def gather_wait(n, b):
        pltpu.make_async_copy(table_hbm.at[idx_v.at[n]], ring.at[b],
                              sem_g.at[b]).wait()

    def store(n, b):
        return pltpu.async_copy(ring.at[b],
                                out_hbm.at[pl.ds(base + n * CHUNK, CHUNK)],
                                sem_s.at[b])

    def store_wait(n, b):
        pltpu.make_async_copy(ring.at[b],
                              out_hbm.at[pl.ds(base + n * CHUNK, CHUNK)],
                              sem_s.at[b]).wait()

    def step(n, b, wait_old_store, prefetch):
        gather_wait(n, b)
        store(n, b)
        b2 = (b + 2) % NBUF
        if wait_old_store:
            store_wait(n - 2, b2)
        if prefetch:
            gather(n + 2, b2)

    gather(0, 0)
    gather(1, 1)

    for b in range(NBUF):
        step(b, b, wait_old_store=(b >= 2), prefetch=True)

    def body(p, carry):
        n0 = p * NBUF
        for b in range(NBUF):
            step(n0 + b, b, wait_old_store=True, prefetch=True)
        return carry

    lax.fori_loop(1, N_CHUNKS // NBUF - 1, body, 0)

    n0 = N_CHUNKS - NBUF
    for b in range(NBUF):
        step(n0 + b, b, wait_old_store=True, prefetch=(b < 2))

    store_wait(N_CHUNKS - 2, (N_CHUNKS - 2) % NBUF)
    store_wait(N_CHUNKS - 1, (N_CHUNKS - 1) % NBUF)


def kernel(indices, table):
    idx = indices.reshape(NW, N_CHUNKS, CHUNK).astype(jnp.int32)
    out = _embed(table, idx)
    return out.reshape(ROWS, COLS, D)


# speedup vs baseline: 1.3748x; 1.3748x over previous
import functools

import jax
import jax.numpy as jnp
from jax import lax
from jax.experimental import pallas as pl
from jax.experimental.pallas import tpu as pltpu
from jax.experimental.pallas import tpu_sc as plsc

ROWS, COLS = 4096, 50
D = 768
V = 64
NC, NS = 2, 16
NW = NC * NS
R_PER_W = ROWS // NW
NBUF = 2

_mesh = plsc.VectorSubcoreMesh(core_axis_name="c", subcore_axis_name="s")


@functools.partial(
    pl.kernel,
    mesh=_mesh,
    out_type=jax.ShapeDtypeStruct((ROWS, COLS, D), jnp.float32),
    scratch_types=[
        pltpu.VMEM((R_PER_W, COLS), jnp.int32),
        pltpu.VMEM((NBUF, COLS, D), jnp.float32),
        pltpu.SemaphoreType.DMA(NBUF),
        pltpu.SemaphoreType.DMA(NBUF),
    ],
)
def _embed(table_hbm, idx_hbm, out_hbm, idx_v, ring, sem_g, sem_s):
    cid = lax.axis_index("c")
    sid = lax.axis_index("s")
    wid = sid * NC + cid
    base = wid * R_PER_W

    pltpu.sync_copy(idx_hbm.at[pl.ds(base, R_PER_W)], idx_v)

    def gather(n, b):
        pltpu.async_copy(table_hbm.at[idx_v.at[n]], ring.at[b], sem_g.at[b])

    def gather_wait(n, b):
        pltpu.make_async_copy(table_hbm.at[idx_v.at[n]], ring.at[b],
                              sem_g.at[b]).wait()

    def store(n, b):
        pltpu.async_copy(ring.at[b], out_hbm.at[base + n], sem_s.at[b])

    def store_wait(n, b):
        pltpu.make_async_copy(ring.at[b], out_hbm.at[base + n],
                              sem_s.at[b]).wait()

    def step(n, b, wait_old_store, prefetch):
        gather_wait(n, b)
        store(n, b)
        b2 = 1 - b
        if wait_old_store:
            store_wait(n - 1, b2)
        if prefetch:
            gather(n + 1, b2)

    gather(0, 0)

    step(0, 0, wait_old_store=False, prefetch=True)

    def body(q, carry):
        n = 1 + q * 2
        step(n, 1, wait_old_store=True, prefetch=True)
        step(n + 1, 0, wait_old_store=True, prefetch=True)
        return carry

    lax.fori_loop(0, 63, body, 0)

    step(127, 1, wait_old_store=True, prefetch=False)

    store_wait(127, 1)


def kernel(indices, table):
    return _embed(table, indices.astype(jnp.int32))
